# Initial kernel scaffold; baseline (speedup 1.0000x reference)
import jax
import jax.numpy as jnp
from jax.experimental import pallas as pl


def kernel(x, edge_index, edge_weight, W, b):
    raise NotImplementedError("write your pallas kernel here")


# speedup vs baseline: 6.5882x; 6.5882x over previous
import functools

import jax
import jax.numpy as jnp
from jax import lax
from jax.experimental import pallas as pl
from jax.experimental.pallas import tpu as pltpu
from jax.experimental.pallas import tpu_sc as plsc

NC = 2
NS = 16
NW = NC * NS
LANES = 16
CHUNK = 128


def _sc_partials(x, src, dst):
    n, d = x.shape
    e = src.shape[0]
    n_chunks = e // CHUNK
    assert e % CHUNK == 0
    ROWS_A = (n // NS) // 8 * 8
    ROWS_LAST = n - (NS - 1) * ROWS_A
    LAST_BASE = (NS - 1) * ROWS_A
    n_acc = ((n + NS * CHUNK - 1) // (NS * CHUNK)) * (NS * CHUNK)
    zchunks = n_acc // (NS * CHUNK)
    outer = (n_chunks + NW - 1) // NW

    mesh = plsc.VectorSubcoreMesh(core_axis_name="c", subcore_axis_name="s")

    @functools.partial(
        pl.kernel,
        mesh=mesh,
        out_type=jax.ShapeDtypeStruct((NC, n, d), jnp.float32),
        scratch_types=[
            pltpu.VMEM_SHARED((n_acc, d), jnp.float32),
            pltpu.VMEM((CHUNK,), jnp.int32),
            pltpu.VMEM((CHUNK,), jnp.int32),
            pltpu.VMEM((CHUNK, d), jnp.float32),
            pltpu.SemaphoreType.DMA,
        ],
    )
    def sc_kernel(x_hbm, src_hbm, dst_hbm, out_hbm, acc, src_v, dst_v, rows_v, sem):
        c = lax.axis_index("c")
        s = lax.axis_index("s")
        wid = s * NC + c

        @pl.when(c == 0)
        def _():
            base = pl.multiple_of(s * ROWS_A, 8)

            @pl.when(s < NS - 1)
            def _():
                pltpu.sync_copy(
                    x_hbm.at[pl.ds(base, ROWS_A)],
                    acc.at[pl.ds(base, ROWS_A)],
                )

            @pl.when(s == NS - 1)
            def _():
                pltpu.sync_copy(
                    x_hbm.at[pl.ds(LAST_BASE, ROWS_LAST)],
                    acc.at[pl.ds(LAST_BASE, ROWS_LAST)],
                )

        @pl.when(c != 0)
        def _():
            @pl.loop(0, CHUNK)
            def _(i):
                @pl.loop(0, d // LANES)
                def _(j):
                    rows_v[i, pl.ds(j * LANES, LANES)] = jnp.zeros(
                        (LANES,), jnp.float32
                    )

            @pl.loop(0, zchunks)
            def _(k):
                pltpu.sync_copy(
                    rows_v, acc.at[pl.ds((s * zchunks + k) * CHUNK, CHUNK)]
                )

        plsc.subcore_barrier()

        @pl.loop(0, outer)
        def _(j):
            chunk = wid + j * NW

            @pl.when(chunk < n_chunks)
            def _():
                base = chunk * CHUNK
                pltpu.sync_copy(src_hbm.at[pl.ds(base, CHUNK)], src_v)
                pltpu.sync_copy(dst_hbm.at[pl.ds(base, CHUNK)], dst_v)
                pltpu.async_copy(x_hbm.at[src_v], rows_v, sem).wait()
                pltpu.sync_copy(rows_v, acc.at[dst_v], add=True)

        plsc.subcore_barrier()

        wbase = pl.multiple_of(s * ROWS_A, 8)

        @pl.when(s < NS - 1)
        def _():
            pltpu.sync_copy(
                acc.at[pl.ds(wbase, ROWS_A)],
                out_hbm.at[c].at[pl.ds(wbase, ROWS_A)],
            )

        @pl.when(s == NS - 1)
        def _():
            pltpu.sync_copy(
                acc.at[pl.ds(LAST_BASE, ROWS_LAST)],
                out_hbm.at[c].at[pl.ds(LAST_BASE, ROWS_LAST)],
            )

    return sc_kernel(x, src, dst)


def _tc_linear(p0, p1, wt, b):
    n, d = p0.shape
    tm = 1000
    assert n % tm == 0

    def mm_kernel(p0_ref, p1_ref, wt_ref, b_ref, o_ref):
        h = p0_ref[...] + p1_ref[...]
        o_ref[...] = (
            jnp.dot(h, wt_ref[...], preferred_element_type=jnp.float32)
            + b_ref[...]
        )

    return pl.pallas_call(
        mm_kernel,
        grid=(n // tm,),
        in_specs=[
            pl.BlockSpec((tm, d), lambda i: (i, 0)),
            pl.BlockSpec((tm, d), lambda i: (i, 0)),
            pl.BlockSpec((d, d), lambda i: (0, 0)),
            pl.BlockSpec((1, d), lambda i: (0, 0)),
        ],
        out_specs=pl.BlockSpec((tm, d), lambda i: (i, 0)),
        out_shape=jax.ShapeDtypeStruct((n, d), jnp.float32),
    )(p0, p1, wt, b.reshape(1, d))


def kernel(x, edge_index, edge_weight, W, b):
    src = edge_index[0].astype(jnp.int32)
    dst = edge_index[1].astype(jnp.int32)
    partials = _sc_partials(x, src, dst)
    return _tc_linear(partials[0], partials[1], W.T, b)
